# baseline (speedup 1.0000x reference)
import jax
import jax.numpy as jnp
from jax import lax
from jax.experimental import pallas as pl
from jax.experimental.pallas import tpu as pltpu

S = 128
N = 4096
D = 64
C = 8
NBUF = 3
K = (S - 1 + C - 1) // C
EPS = 1e-12


def _gumbel(u):
    return -jnp.log(-jnp.log(u + EPS) + EPS)


def _isir_body(tc_ref, wc_ref, prop_ref, gum_ref, m_ref,
               traj_ref, w_out, i_out, sel_out,
               buf_ref, best_ref, in_sems, out_sems, sem0):
    k = pl.program_id(0)

    def chunk_start(kk):
        return jnp.minimum(kk * C, (S - 1) - C)

    H = C // 2

    class _Pair:
        def __init__(self, copies):
            self.copies = copies

        def start(self):
            for c in self.copies:
                c.start()

        def wait(self):
            for c in self.copies:
                c.wait()

    def in_copy(kk):
        b = lax.rem(kk, NBUF)
        s = chunk_start(kk)
        return _Pair([
            pltpu.make_async_copy(
                prop_ref.at[pl.ds(s + h * H, H)],
                buf_ref.at[b, pl.ds(h * H, H)],
                in_sems.at[b, h])
            for h in range(2)])

    def out_copy(kk):
        b = lax.rem(kk, NBUF)
        s = chunk_start(kk)
        return _Pair([
            pltpu.make_async_copy(
                buf_ref.at[b, pl.ds(h * H, H)],
                traj_ref.at[pl.ds(s + 1 + h * H, H)],
                out_sems.at[b, h])
            for h in range(2)])

    @pl.when(k == 0)
    def _():
        pltpu.make_async_copy(tc_ref, traj_ref.at[0], sem0).start()
        in_copy(0).start()
        in_copy(1).start()
        w0 = wc_ref[...]
        w_out[0] = w0
        best_ref[...] = w0 + _gumbel(gum_ref[0])
        i_out[...] = jnp.zeros((1, N), jnp.int32)
        sel_out[...] = tc_ref[...]

    in_copy(k).wait()
    out_copy(k).start()

    m = m_ref[...]
    half_m2 = 0.5 * jnp.sum(m * m)
    start = chunk_start(k)
    best = best_ref[...]
    ibest = i_out[...]
    bslot = lax.rem(k, NBUF)
    for j in range(C):
        slot = start + 1 + j
        t = buf_ref[bslot, j]
        w = jnp.sum(t * m, axis=0)[None, :] - half_m2
        w_out[slot] = w
        score = w + _gumbel(gum_ref[slot])
        win = score > best
        best = jnp.where(win, score, best)
        ibest = jnp.where(win, slot, ibest)

        @pl.when(jnp.any(win))
        def _():
            sel_out[...] = jnp.where(win, t, sel_out[...])

    best_ref[...] = best
    i_out[...] = ibest

    @pl.when((k >= 1) & (k + 2 < K))
    def _():
        out_copy(k - 1).wait()

    @pl.when(k + 2 < K)
    def _():
        in_copy(k + 2).start()

    @pl.when(k == K - 1)
    def _():
        out_copy(K - 3).wait()
        out_copy(K - 2).wait()
        out_copy(K - 1).wait()
        pltpu.make_async_copy(tc_ref, traj_ref.at[0], sem0).wait()


def _isir_pass(tct, wc2, props_t, gu3, m2):
    return pl.pallas_call(
        _isir_body,
        grid=(K,),
        in_specs=[
            pl.BlockSpec(memory_space=pltpu.VMEM),
            pl.BlockSpec(memory_space=pltpu.VMEM),
            pl.BlockSpec(memory_space=pl.ANY),
            pl.BlockSpec(memory_space=pltpu.VMEM),
            pl.BlockSpec(memory_space=pltpu.VMEM),
        ],
        out_specs=[
            pl.BlockSpec(memory_space=pl.ANY),
            pl.BlockSpec(memory_space=pltpu.VMEM),
            pl.BlockSpec(memory_space=pltpu.VMEM),
            pl.BlockSpec(memory_space=pltpu.VMEM),
        ],
        out_shape=[
            jax.ShapeDtypeStruct((S, D, N), jnp.float32),
            jax.ShapeDtypeStruct((S, 1, N), jnp.float32),
            jax.ShapeDtypeStruct((1, N), jnp.int32),
            jax.ShapeDtypeStruct((D, N), jnp.float32),
        ],
        scratch_shapes=[
            pltpu.VMEM((NBUF, C, D, N), jnp.float32),
            pltpu.VMEM((1, N), jnp.float32),
            pltpu.SemaphoreType.DMA((NBUF, 2)),
            pltpu.SemaphoreType.DMA((NBUF, 2)),
            pltpu.SemaphoreType.DMA,
        ],
    )(tct, wc2, props_t, gu3, m2)


def kernel(traj_cur, weights_cur, proposals, gumbel_u, target_mean):
    tct = traj_cur.T
    wc2 = weights_cur.reshape(1, N)
    props_t = proposals.transpose(0, 2, 1)
    gu3 = gumbel_u.reshape(S, 1, N)
    m2 = target_mean.reshape(D, 1)
    traj_tot_t, w3, i2, sel_t = _isir_pass(tct, wc2, props_t, gu3, m2)
    traj_tot = traj_tot_t.transpose(0, 2, 1)
    weights_tot = w3.reshape(S, N)
    i = i2.reshape(N)
    traj_sel = sel_t.T
    return (traj_tot, weights_tot, i, traj_sel)


# speedup vs baseline: 1.0103x; 1.0103x over previous
import jax
import jax.numpy as jnp
from jax import lax
from jax.experimental import pallas as pl
from jax.experimental.pallas import tpu as pltpu

S = 128
N = 4096
D = 64
C = 8
NBUF = 4
K = (S - 1 + C - 1) // C
EPS = 1e-12


def _gumbel(u):
    return -jnp.log(-jnp.log(u + EPS) + EPS)


def _isir_body(tc_ref, wc_ref, prop_ref, gum_ref, m_ref,
               traj_ref, w_out, i_out, sel_out,
               buf_ref, best_ref, in_sems, out_sems, sem0):
    k = pl.program_id(0)

    def chunk_start(kk):
        return jnp.minimum(kk * C, (S - 1) - C)

    def in_copy(kk):
        return pltpu.make_async_copy(
            prop_ref.at[pl.ds(chunk_start(kk), C)],
            buf_ref.at[lax.rem(kk, NBUF)],
            in_sems.at[lax.rem(kk, NBUF)])

    def out_copy(kk):
        return pltpu.make_async_copy(
            buf_ref.at[lax.rem(kk, NBUF)],
            traj_ref.at[pl.ds(chunk_start(kk) + 1, C)],
            out_sems.at[lax.rem(kk, NBUF)])

    @pl.when(k == 0)
    def _():
        pltpu.make_async_copy(tc_ref, traj_ref.at[0], sem0).start()
        in_copy(0).start()
        in_copy(1).start()
        in_copy(2).start()
        w0 = wc_ref[...]
        w_out[0] = w0
        best_ref[...] = w0 + _gumbel(gum_ref[0])
        i_out[...] = jnp.zeros((1, N), jnp.int32)
        sel_out[...] = tc_ref[...]

    in_copy(k).wait()
    out_copy(k).start()

    m = m_ref[...]
    half_m2 = 0.5 * jnp.sum(m * m)
    start = chunk_start(k)
    best = best_ref[...]
    ibest = i_out[...]
    bslot = lax.rem(k, NBUF)
    for j in range(C):
        slot = start + 1 + j
        t = buf_ref[bslot, j]
        w = jnp.sum(t * m, axis=0)[None, :] - half_m2
        w_out[slot] = w
        score = w + _gumbel(gum_ref[slot])
        win = score > best
        best = jnp.where(win, score, best)
        ibest = jnp.where(win, slot, ibest)

        @pl.when(jnp.any(win))
        def _():
            sel_out[...] = jnp.where(win, t, sel_out[...])

    best_ref[...] = best
    i_out[...] = ibest

    @pl.when((k >= 1) & (k + 3 < K))
    def _():
        out_copy(k - 1).wait()

    @pl.when(k + 3 < K)
    def _():
        in_copy(k + 3).start()

    @pl.when(k == K - 1)
    def _():
        out_copy(K - 4).wait()
        out_copy(K - 3).wait()
        out_copy(K - 2).wait()
        out_copy(K - 1).wait()
        pltpu.make_async_copy(tc_ref, traj_ref.at[0], sem0).wait()


def _isir_pass(tct, wc2, props_t, gu3, m2):
    return pl.pallas_call(
        _isir_body,
        grid=(K,),
        in_specs=[
            pl.BlockSpec(memory_space=pltpu.VMEM),
            pl.BlockSpec(memory_space=pltpu.VMEM),
            pl.BlockSpec(memory_space=pl.ANY),
            pl.BlockSpec(memory_space=pltpu.VMEM),
            pl.BlockSpec(memory_space=pltpu.VMEM),
        ],
        out_specs=[
            pl.BlockSpec(memory_space=pl.ANY),
            pl.BlockSpec(memory_space=pltpu.VMEM),
            pl.BlockSpec(memory_space=pltpu.VMEM),
            pl.BlockSpec(memory_space=pltpu.VMEM),
        ],
        out_shape=[
            jax.ShapeDtypeStruct((S, D, N), jnp.float32),
            jax.ShapeDtypeStruct((S, 1, N), jnp.float32),
            jax.ShapeDtypeStruct((1, N), jnp.int32),
            jax.ShapeDtypeStruct((D, N), jnp.float32),
        ],
        scratch_shapes=[
            pltpu.VMEM((NBUF, C, D, N), jnp.float32),
            pltpu.VMEM((1, N), jnp.float32),
            pltpu.SemaphoreType.DMA((NBUF,)),
            pltpu.SemaphoreType.DMA((NBUF,)),
            pltpu.SemaphoreType.DMA,
        ],
    )(tct, wc2, props_t, gu3, m2)


def kernel(traj_cur, weights_cur, proposals, gumbel_u, target_mean):
    tct = traj_cur.T
    wc2 = weights_cur.reshape(1, N)
    props_t = proposals.transpose(0, 2, 1)
    gu3 = gumbel_u.reshape(S, 1, N)
    m2 = target_mean.reshape(D, 1)
    traj_tot_t, w3, i2, sel_t = _isir_pass(tct, wc2, props_t, gu3, m2)
    traj_tot = traj_tot_t.transpose(0, 2, 1)
    weights_tot = w3.reshape(S, N)
    i = i2.reshape(N)
    traj_sel = sel_t.T
    return (traj_tot, weights_tot, i, traj_sel)


# speedup vs baseline: 1.0109x; 1.0006x over previous
import jax
import jax.numpy as jnp
from jax import lax
from jax.experimental import pallas as pl
from jax.experimental.pallas import tpu as pltpu

S = 128
N = 4096
D = 64
C = 8
NBUF = 6
K = (S - 1 + C - 1) // C
EPS = 1e-12


def _gumbel(u):
    return -jnp.log(-jnp.log(u + EPS) + EPS)


def _isir_body(tc_ref, wc_ref, prop_ref, gum_ref, m_ref,
               traj_ref, w_out, i_out, sel_out,
               buf_ref, best_ref, in_sems, out_sems, sem0):
    k = pl.program_id(0)

    def chunk_start(kk):
        return jnp.minimum(kk * C, (S - 1) - C)

    def in_copy(kk):
        return pltpu.make_async_copy(
            prop_ref.at[pl.ds(chunk_start(kk), C)],
            buf_ref.at[lax.rem(kk, NBUF)],
            in_sems.at[lax.rem(kk, NBUF)])

    def out_copy(kk):
        return pltpu.make_async_copy(
            buf_ref.at[lax.rem(kk, NBUF)],
            traj_ref.at[pl.ds(chunk_start(kk) + 1, C)],
            out_sems.at[lax.rem(kk, NBUF)])

    @pl.when(k == 0)
    def _():
        pltpu.make_async_copy(tc_ref, traj_ref.at[0], sem0).start()
        for kk in range(NBUF - 1):
            in_copy(kk).start()
        w0 = wc_ref[...]
        w_out[0] = w0
        best_ref[...] = w0 + _gumbel(gum_ref[0])
        i_out[...] = jnp.zeros((1, N), jnp.int32)
        sel_out[...] = tc_ref[...]

    in_copy(k).wait()
    out_copy(k).start()

    m = m_ref[...]
    half_m2 = 0.5 * jnp.sum(m * m)
    start = chunk_start(k)
    best = best_ref[...]
    ibest = i_out[...]
    bslot = lax.rem(k, NBUF)
    for j in range(C):
        slot = start + 1 + j
        t = buf_ref[bslot, j]
        w = jnp.sum(t * m, axis=0)[None, :] - half_m2
        w_out[slot] = w
        score = w + _gumbel(gum_ref[slot])
        win = score > best
        best = jnp.where(win, score, best)
        ibest = jnp.where(win, slot, ibest)

        @pl.when(jnp.any(win))
        def _():
            sel_out[...] = jnp.where(win, t, sel_out[...])

    best_ref[...] = best
    i_out[...] = ibest

    @pl.when((k >= 1) & (k + NBUF - 1 < K))
    def _():
        out_copy(k - 1).wait()

    @pl.when(k + NBUF - 1 < K)
    def _():
        in_copy(k + NBUF - 1).start()

    @pl.when(k == K - 1)
    def _():
        for kk in range(max(0, K - NBUF), K):
            out_copy(kk).wait()
        pltpu.make_async_copy(tc_ref, traj_ref.at[0], sem0).wait()


def _isir_pass(tct, wc2, props_t, gu3, m2):
    return pl.pallas_call(
        _isir_body,
        grid=(K,),
        in_specs=[
            pl.BlockSpec(memory_space=pltpu.VMEM),
            pl.BlockSpec(memory_space=pltpu.VMEM),
            pl.BlockSpec(memory_space=pl.ANY),
            pl.BlockSpec(memory_space=pltpu.VMEM),
            pl.BlockSpec(memory_space=pltpu.VMEM),
        ],
        out_specs=[
            pl.BlockSpec(memory_space=pl.ANY),
            pl.BlockSpec(memory_space=pltpu.VMEM),
            pl.BlockSpec(memory_space=pltpu.VMEM),
            pl.BlockSpec(memory_space=pltpu.VMEM),
        ],
        out_shape=[
            jax.ShapeDtypeStruct((S, D, N), jnp.float32),
            jax.ShapeDtypeStruct((S, 1, N), jnp.float32),
            jax.ShapeDtypeStruct((1, N), jnp.int32),
            jax.ShapeDtypeStruct((D, N), jnp.float32),
        ],
        scratch_shapes=[
            pltpu.VMEM((NBUF, C, D, N), jnp.float32),
            pltpu.VMEM((1, N), jnp.float32),
            pltpu.SemaphoreType.DMA((NBUF,)),
            pltpu.SemaphoreType.DMA((NBUF,)),
            pltpu.SemaphoreType.DMA,
        ],
    )(tct, wc2, props_t, gu3, m2)


def kernel(traj_cur, weights_cur, proposals, gumbel_u, target_mean):
    tct = traj_cur.T
    wc2 = weights_cur.reshape(1, N)
    props_t = proposals.transpose(0, 2, 1)
    gu3 = gumbel_u.reshape(S, 1, N)
    m2 = target_mean.reshape(D, 1)
    traj_tot_t, w3, i2, sel_t = _isir_pass(tct, wc2, props_t, gu3, m2)
    traj_tot = traj_tot_t.transpose(0, 2, 1)
    weights_tot = w3.reshape(S, N)
    i = i2.reshape(N)
    traj_sel = sel_t.T
    return (traj_tot, weights_tot, i, traj_sel)
